# Initial kernel scaffold; baseline (speedup 1.0000x reference)
import jax
import jax.numpy as jnp
from jax.experimental import pallas as pl


def kernel(scale, vec, omegas_array, score_norms):
    raise NotImplementedError("write your pallas kernel here")


# speedup vs baseline: 42.9727x; 42.9727x over previous
import functools
import numpy as np
import jax
import jax.numpy as jnp
from jax import lax
from jax.experimental import pallas as pl
from jax.experimental.pallas import tpu as pltpu
from jax.experimental.pallas import tpu_sc as plsc

_MIN_EPS = 0.01
_MAX_EPS = 2.0
_N_EPS = 1000
_N_OM = 1000

_NC, _NS = 2, 16
_NW = _NC * _NS
_CN = 2048


def _sc_sumsq_body(vec_hbm, s_hbm, vbuf, sbuf, nsub):
    wid = lax.axis_index("s") * _NC + lax.axis_index("c")
    base = wid * (nsub * _CN)
    lanes = lax.iota(jnp.int32, 16)

    def sub(k, _):
        row0 = base + k * _CN
        pltpu.sync_copy(vec_hbm.at[pl.ds(3 * row0, 3 * _CN)], vbuf)

        def it(i, _):
            i3 = (lanes + i * 16) * 3
            x = plsc.load_gather(vbuf, [i3])
            y = plsc.load_gather(vbuf, [i3 + 1])
            z = plsc.load_gather(vbuf, [i3 + 2])
            sbuf[pl.ds(i * 16, 16)] = (x * x + y * y) + z * z
            return 0

        lax.fori_loop(0, _CN // 16, it, 0)
        pltpu.sync_copy(sbuf, s_hbm.at[pl.ds(row0, _CN)])
        return 0

    lax.fori_loop(0, nsub, sub, 0)


def _tc_body(scale_ref, s_ref, om_ref, grow_ref):
    eps = scale_ref[...]
    fi = (jnp.log10(eps) - np.log10(_MIN_EPS)) / (
        np.log10(_MAX_EPS) - np.log10(_MIN_EPS)) * _N_EPS
    ei = jnp.clip(jnp.round(fi).astype(jnp.int32), 0, _N_EPS - 1)
    grow_ref[...] = ei * _N_OM
    om_ref[...] = jnp.sqrt(s_ref[...])


def _sc_main_body(om_hbm, grow_hbm, vec_hbm, omg_hbm, tab_hbm, out_hbm,
                  ombuf, gbuf, vbuf, obuf, tbuf, cxbuf, cybuf, czbuf,
                  g0buf, g1buf, f0buf, f1buf, omg, sem, nsub, inv_h, om0):
    wid = lax.axis_index("s") * _NC + lax.axis_index("c")
    base = wid * (nsub * _CN)
    lanes = lax.iota(jnp.int32, 16)
    pltpu.sync_copy(omg_hbm, omg)

    def sub(k, _):
        row0 = base + k * _CN
        pltpu.sync_copy(om_hbm.at[pl.ds(row0, _CN)], ombuf)
        pltpu.sync_copy(grow_hbm.at[pl.ds(row0, _CN)], gbuf)
        pltpu.sync_copy(vec_hbm.at[pl.ds(3 * row0, 3 * _CN)], vbuf)

        def it1(i, _):
            sl = pl.ds(i * 16, 16)
            i3 = (lanes + i * 16) * 3
            x = plsc.load_gather(vbuf, [i3])
            y = plsc.load_gather(vbuf, [i3 + 1])
            z = plsc.load_gather(vbuf, [i3 + 2])
            om = ombuf[sl]
            grow = gbuf[sl]
            pos = (om - om0) * inv_h
            c = jnp.clip(pos.astype(jnp.int32) + 1, 1, _N_OM - 1)
            for _r in range(3):
                w1 = plsc.load_gather(omg, [c])
                w0 = plsc.load_gather(omg, [c - 1])
                up = (w1 < om) & (c < _N_OM - 1)
                dn = (w0 >= om) & (c > 1)
                c = jnp.where(up, c + 1, jnp.where(dn, c - 1, c))
            x1 = plsc.load_gather(omg, [c])
            x0 = plsc.load_gather(omg, [c - 1])
            t = (om - x0) / (x1 - x0)
            inv = 1.0 / om
            g0 = grow + c - 1
            tbuf[sl] = t
            g0buf[sl] = g0
            g1buf[sl] = g0 + 1
            cxbuf[sl] = x * inv
            cybuf[sl] = y * inv
            czbuf[sl] = z * inv
            return 0

        lax.fori_loop(0, _CN // 16, it1, 0)
        pltpu.async_copy(tab_hbm.at[g0buf], f0buf, sem).wait()
        pltpu.async_copy(tab_hbm.at[g1buf], f1buf, sem).wait()

        def it2(i, _):
            sl = pl.ds(i * 16, 16)
            i3 = (lanes + i * 16) * 3
            f0 = f0buf[sl]
            f1 = f1buf[sl]
            interp = f0 + (f1 - f0) * tbuf[sl]
            plsc.store_scatter(obuf, [i3], interp * cxbuf[sl])
            plsc.store_scatter(obuf, [i3 + 1], interp * cybuf[sl])
            plsc.store_scatter(obuf, [i3 + 2], interp * czbuf[sl])
            return 0

        lax.fori_loop(0, _CN // 16, it2, 0)
        pltpu.sync_copy(obuf, out_hbm.at[pl.ds(3 * row0, 3 * _CN)])
        return 0

    lax.fori_loop(0, nsub, sub, 0)


def kernel(scale, vec, omegas_array, score_norms):
    b = scale.shape[0]
    assert b % (_NW * _CN) == 0
    nsub = b // (_NW * _CN)
    vec_flat = vec.reshape(3 * b)
    tab_flat = score_norms.reshape(_N_EPS * _N_OM)

    mesh = plsc.VectorSubcoreMesh(core_axis_name="c", subcore_axis_name="s")
    sc_params = pltpu.CompilerParams(needs_layout_passes=False)

    sumsq = pl.kernel(
        functools.partial(_sc_sumsq_body, nsub=nsub),
        out_type=jax.ShapeDtypeStruct((b,), jnp.float32),
        mesh=mesh,
        scratch_types=[
            pltpu.VMEM((3 * _CN,), jnp.float32),
            pltpu.VMEM((_CN,), jnp.float32),
        ],
        compiler_params=sc_params,
    )
    s = sumsq(vec_flat)

    rows, cols = 512, b // 512
    om2, grow2 = pl.pallas_call(
        _tc_body,
        grid=(8,),
        in_specs=[
            pl.BlockSpec((rows // 8, cols), lambda i: (i, 0)),
            pl.BlockSpec((rows // 8, cols), lambda i: (i, 0)),
        ],
        out_specs=[
            pl.BlockSpec((rows // 8, cols), lambda i: (i, 0)),
            pl.BlockSpec((rows // 8, cols), lambda i: (i, 0)),
        ],
        out_shape=[
            jax.ShapeDtypeStruct((rows, cols), jnp.float32),
            jax.ShapeDtypeStruct((rows, cols), jnp.int32),
        ],
    )(scale.reshape(rows, cols), s.reshape(rows, cols))
    om = om2.reshape(b)
    grow = grow2.reshape(b)

    h = (np.pi - 1e-3) / (_N_OM - 1)
    main = pl.kernel(
        functools.partial(_sc_main_body, nsub=nsub,
                          inv_h=np.float32(1.0 / h), om0=np.float32(1e-3)),
        out_type=jax.ShapeDtypeStruct((3 * b,), jnp.float32),
        mesh=mesh,
        scratch_types=[
            pltpu.VMEM((_CN,), jnp.float32),
            pltpu.VMEM((_CN,), jnp.int32),
            pltpu.VMEM((3 * _CN,), jnp.float32),
            pltpu.VMEM((3 * _CN,), jnp.float32),
            pltpu.VMEM((_CN,), jnp.float32),
            pltpu.VMEM((_CN,), jnp.float32),
            pltpu.VMEM((_CN,), jnp.float32),
            pltpu.VMEM((_CN,), jnp.float32),
            pltpu.VMEM((_CN,), jnp.int32),
            pltpu.VMEM((_CN,), jnp.int32),
            pltpu.VMEM((_CN,), jnp.float32),
            pltpu.VMEM((_CN,), jnp.float32),
            pltpu.VMEM((_N_OM,), jnp.float32),
            pltpu.SemaphoreType.DMA,
        ],
        compiler_params=sc_params,
    )
    out_flat = main(om, grow, vec_flat, omegas_array, tab_flat)
    return out_flat.reshape(b, 3).astype(scale.dtype)


# speedup vs baseline: 371.1348x; 8.6365x over previous
import functools
import numpy as np
import jax
import jax.numpy as jnp
from jax import lax
from jax.experimental import pallas as pl
from jax.experimental.pallas import tpu as pltpu
from jax.experimental.pallas import tpu_sc as plsc

_MIN_EPS = 0.01
_MAX_EPS = 2.0
_N_EPS = 1000
_N_OM = 1000

_NC, _NS = 2, 16
_NW = _NC * _NS
_CN = 8192


def _tc_body(scale_ref, x_ref, y_ref, z_ref, om_ref, grow_ref):
    eps = scale_ref[...]
    fi = (jnp.log10(eps) - np.log10(_MIN_EPS)) / (
        np.log10(_MAX_EPS) - np.log10(_MIN_EPS)) * _N_EPS
    ei = jnp.clip(jnp.round(fi).astype(jnp.int32), 0, _N_EPS - 1)
    grow_ref[...] = ei * _N_OM
    x = x_ref[...]
    y = y_ref[...]
    z = z_ref[...]
    om_ref[...] = jnp.sqrt(x * x + y * y + z * z)


def _sc_main_body(om_hbm, grow_hbm, x_hbm, y_hbm, z_hbm, omg_hbm, tab_hbm,
                  ox_hbm, oy_hbm, oz_hbm,
                  ombuf, gbuf, xbuf, ybuf, zbuf, tbuf, ibuf,
                  g0buf, g1buf, f0buf, f1buf, omg, sem, nsub, inv_h, om0):
    wid = lax.axis_index("s") * _NC + lax.axis_index("c")
    base = wid * (nsub * _CN)
    pltpu.sync_copy(omg_hbm, omg)

    def sub(k, _):
        row0 = base + k * _CN
        cs = pl.ds(row0, _CN)
        pltpu.sync_copy(om_hbm.at[cs], ombuf)
        pltpu.sync_copy(grow_hbm.at[cs], gbuf)
        pltpu.sync_copy(x_hbm.at[cs], xbuf)
        pltpu.sync_copy(y_hbm.at[cs], ybuf)
        pltpu.sync_copy(z_hbm.at[cs], zbuf)

        def it1(i, _):
            sl = pl.ds(i * 16, 16)
            om = ombuf[sl]
            grow = gbuf[sl]
            pos = (om - om0) * inv_h
            c = jnp.clip(pos.astype(jnp.int32) + 1, 1, _N_OM - 1)
            for _r in range(3):
                w1 = plsc.load_gather(omg, [c])
                w0 = plsc.load_gather(omg, [c - 1])
                up = (w1 < om) & (c < _N_OM - 1)
                dn = (w0 >= om) & (c > 1)
                c = jnp.where(up, c + 1, jnp.where(dn, c - 1, c))
            x1 = plsc.load_gather(omg, [c])
            x0 = plsc.load_gather(omg, [c - 1])
            g0 = grow + c - 1
            tbuf[sl] = (om - x0) / (x1 - x0)
            ibuf[sl] = 1.0 / om
            g0buf[sl] = g0
            g1buf[sl] = g0 + 1
            return 0

        lax.fori_loop(0, _CN // 16, it1, 0)
        pltpu.async_copy(tab_hbm.at[g0buf], f0buf, sem).wait()
        pltpu.async_copy(tab_hbm.at[g1buf], f1buf, sem).wait()

        def it2(i, _):
            sl = pl.ds(i * 16, 16)
            f0 = f0buf[sl]
            f1 = f1buf[sl]
            q = (f0 + (f1 - f0) * tbuf[sl]) * ibuf[sl]
            xbuf[sl] = q * xbuf[sl]
            ybuf[sl] = q * ybuf[sl]
            zbuf[sl] = q * zbuf[sl]
            return 0

        lax.fori_loop(0, _CN // 16, it2, 0)
        pltpu.sync_copy(xbuf, ox_hbm.at[cs])
        pltpu.sync_copy(ybuf, oy_hbm.at[cs])
        pltpu.sync_copy(zbuf, oz_hbm.at[cs])
        return 0

    lax.fori_loop(0, nsub, sub, 0)


def kernel(scale, vec, omegas_array, score_norms):
    b = scale.shape[0]
    assert b % (_NW * _CN) == 0
    nsub = b // (_NW * _CN)
    xs = vec[:, 0]
    ys = vec[:, 1]
    zs = vec[:, 2]
    tab_flat = score_norms.reshape(_N_EPS * _N_OM)

    grid = 16
    bs = b // grid
    om, grow = pl.pallas_call(
        _tc_body,
        grid=(grid,),
        in_specs=[pl.BlockSpec((bs,), lambda i: (i,))] * 4,
        out_specs=[pl.BlockSpec((bs,), lambda i: (i,))] * 2,
        out_shape=[
            jax.ShapeDtypeStruct((b,), jnp.float32),
            jax.ShapeDtypeStruct((b,), jnp.int32),
        ],
    )(scale, xs, ys, zs)

    h = (np.pi - 1e-3) / (_N_OM - 1)
    mesh = plsc.VectorSubcoreMesh(core_axis_name="c", subcore_axis_name="s")
    main = pl.kernel(
        functools.partial(_sc_main_body, nsub=nsub,
                          inv_h=np.float32(1.0 / h), om0=np.float32(1e-3)),
        out_type=[jax.ShapeDtypeStruct((b,), jnp.float32)] * 3,
        mesh=mesh,
        scratch_types=[
            pltpu.VMEM((_CN,), jnp.float32),
            pltpu.VMEM((_CN,), jnp.int32),
            pltpu.VMEM((_CN,), jnp.float32),
            pltpu.VMEM((_CN,), jnp.float32),
            pltpu.VMEM((_CN,), jnp.float32),
            pltpu.VMEM((_CN,), jnp.float32),
            pltpu.VMEM((_CN,), jnp.float32),
            pltpu.VMEM((_CN,), jnp.int32),
            pltpu.VMEM((_CN,), jnp.int32),
            pltpu.VMEM((_CN,), jnp.float32),
            pltpu.VMEM((_CN,), jnp.float32),
            pltpu.VMEM((_N_OM,), jnp.float32),
            pltpu.SemaphoreType.DMA,
        ],
        compiler_params=pltpu.CompilerParams(needs_layout_passes=False),
    )
    ox, oy, oz = main(om, grow, xs, ys, zs, omegas_array, tab_flat)
    return jnp.stack([ox, oy, oz], axis=1).astype(scale.dtype)


# speedup vs baseline: 423.2408x; 1.1404x over previous
import functools
import numpy as np
import jax
import jax.numpy as jnp
from jax import lax
from jax.experimental import pallas as pl
from jax.experimental.pallas import tpu as pltpu
from jax.experimental.pallas import tpu_sc as plsc

_MIN_EPS = 0.01
_MAX_EPS = 2.0
_N_EPS = 1000
_N_OM = 1000

_NC, _NS = 2, 16
_NW = _NC * _NS
_CN = 8192


_C1 = np.float32(1.0 / np.log(10.0))
_C2 = np.float32(434.588)
_RNE_MAGIC = np.float32(12582912.0)


def _tc_body(scale_ref, x_ref, y_ref, z_ref, om_ref, grow_ref):
    eps = scale_ref[...]
    fi = (jnp.log(eps) * _C1 + np.float32(2.0)) * _C2
    r = (fi + _RNE_MAGIC) - _RNE_MAGIC
    ei = jnp.clip(r.astype(jnp.int32), 0, _N_EPS - 1)
    grow_ref[...] = ei * _N_OM
    x = x_ref[...]
    y = y_ref[...]
    z = z_ref[...]
    om_ref[...] = jnp.sqrt((x * x + z * z) + y * y)


def _sc_main_body(om_hbm, grow_hbm, x_hbm, y_hbm, z_hbm, omg_hbm, tab_hbm,
                  ox_hbm, oy_hbm, oz_hbm,
                  ombuf, gbuf, xbuf, ybuf, zbuf, tbuf, ibuf,
                  g0buf, g1buf, f0buf, f1buf, omg, sem, nsub, inv_h, om0):
    wid = lax.axis_index("s") * _NC + lax.axis_index("c")
    base = wid * (nsub * _CN)
    pltpu.sync_copy(omg_hbm, omg)

    def sub(k, _):
        row0 = base + k * _CN
        cs = pl.ds(row0, _CN)
        pltpu.sync_copy(om_hbm.at[cs], ombuf)
        pltpu.sync_copy(grow_hbm.at[cs], gbuf)
        pltpu.sync_copy(x_hbm.at[cs], xbuf)
        pltpu.sync_copy(y_hbm.at[cs], ybuf)
        pltpu.sync_copy(z_hbm.at[cs], zbuf)

        def it1(i, _):
            sl = pl.ds(i * 16, 16)
            om = ombuf[sl]
            grow = gbuf[sl]
            pos = (om - om0) * inv_h
            c0 = jnp.clip(pos.astype(jnp.int32), 0, _N_OM - 2)
            w0 = plsc.load_gather(omg, [c0])
            w1 = plsc.load_gather(omg, [c0 + 1])
            j = c0 + (w0 < om).astype(jnp.int32) + (w1 < om).astype(jnp.int32)
            c = jnp.clip(j, 1, _N_OM - 1)
            x1 = plsc.load_gather(omg, [c])
            x0 = plsc.load_gather(omg, [c - 1])
            g0 = grow + c - 1
            tbuf[sl] = (om - x0) / (x1 - x0)
            ibuf[sl] = 1.0 / om
            g0buf[sl] = g0
            g1buf[sl] = g0 + 1
            return 0

        lax.fori_loop(0, _CN // 16, it1, 0)
        pltpu.async_copy(tab_hbm.at[g0buf], f0buf, sem).wait()
        pltpu.async_copy(tab_hbm.at[g1buf], f1buf, sem).wait()

        def it2(i, _):
            sl = pl.ds(i * 16, 16)
            f0 = f0buf[sl]
            f1 = f1buf[sl]
            q = (f0 + (f1 - f0) * tbuf[sl]) * ibuf[sl]
            xbuf[sl] = q * xbuf[sl]
            ybuf[sl] = q * ybuf[sl]
            zbuf[sl] = q * zbuf[sl]
            return 0

        lax.fori_loop(0, _CN // 16, it2, 0)
        pltpu.sync_copy(xbuf, ox_hbm.at[cs])
        pltpu.sync_copy(ybuf, oy_hbm.at[cs])
        pltpu.sync_copy(zbuf, oz_hbm.at[cs])
        return 0

    lax.fori_loop(0, nsub, sub, 0)


def kernel(scale, vec, omegas_array, score_norms):
    b = scale.shape[0]
    assert b % (_NW * _CN) == 0
    nsub = b // (_NW * _CN)
    xs = vec[:, 0]
    ys = vec[:, 1]
    zs = vec[:, 2]
    tab_flat = score_norms.reshape(_N_EPS * _N_OM)

    grid = 16
    bs = b // grid
    om, grow = pl.pallas_call(
        _tc_body,
        grid=(grid,),
        in_specs=[pl.BlockSpec((bs,), lambda i: (i,))] * 4,
        out_specs=[pl.BlockSpec((bs,), lambda i: (i,))] * 2,
        out_shape=[
            jax.ShapeDtypeStruct((b,), jnp.float32),
            jax.ShapeDtypeStruct((b,), jnp.int32),
        ],
    )(scale, xs, ys, zs)

    h = (np.pi - 1e-3) / (_N_OM - 1)
    mesh = plsc.VectorSubcoreMesh(core_axis_name="c", subcore_axis_name="s")
    main = pl.kernel(
        functools.partial(_sc_main_body, nsub=nsub,
                          inv_h=np.float32(1.0 / h), om0=np.float32(1e-3)),
        out_type=[jax.ShapeDtypeStruct((b,), jnp.float32)] * 3,
        mesh=mesh,
        scratch_types=[
            pltpu.VMEM((_CN,), jnp.float32),
            pltpu.VMEM((_CN,), jnp.int32),
            pltpu.VMEM((_CN,), jnp.float32),
            pltpu.VMEM((_CN,), jnp.float32),
            pltpu.VMEM((_CN,), jnp.float32),
            pltpu.VMEM((_CN,), jnp.float32),
            pltpu.VMEM((_CN,), jnp.float32),
            pltpu.VMEM((_CN,), jnp.int32),
            pltpu.VMEM((_CN,), jnp.int32),
            pltpu.VMEM((_CN,), jnp.float32),
            pltpu.VMEM((_CN,), jnp.float32),
            pltpu.VMEM((_N_OM,), jnp.float32),
            pltpu.SemaphoreType.DMA,
        ],
        compiler_params=pltpu.CompilerParams(needs_layout_passes=False),
    )
    ox, oy, oz = main(om, grow, xs, ys, zs, omegas_array, tab_flat)
    return jnp.stack([ox, oy, oz], axis=1).astype(scale.dtype)


# speedup vs baseline: 486.6962x; 1.1499x over previous
import functools
import numpy as np
import jax
import jax.numpy as jnp
from jax import lax
from jax.experimental import pallas as pl
from jax.experimental.pallas import tpu as pltpu
from jax.experimental.pallas import tpu_sc as plsc

_MIN_EPS = 0.01
_MAX_EPS = 2.0
_N_EPS = 1000
_N_OM = 1000

_NC, _NS = 2, 16
_NW = _NC * _NS
_CN = 4096


_C1 = np.float32(1.0 / np.log(10.0))
_C2 = np.float32(434.588)
_RNE_MAGIC = np.float32(12582912.0)


def _tc_body(scale_ref, x_ref, y_ref, z_ref, om_ref, grow_ref):
    eps = scale_ref[...]
    fi = (jnp.log(eps) * _C1 + np.float32(2.0)) * _C2
    r = (fi + _RNE_MAGIC) - _RNE_MAGIC
    ei = jnp.clip(r.astype(jnp.int32), 0, _N_EPS - 1)
    grow_ref[...] = ei * _N_OM
    x = x_ref[...]
    y = y_ref[...]
    z = z_ref[...]
    om_ref[...] = jnp.sqrt((x * x + z * z) + y * y)


def _sc_main_body(om_hbm, grow_hbm, x_hbm, y_hbm, z_hbm, omg_hbm, tab_hbm,
                  ox_hbm, oy_hbm, oz_hbm,
                  ombuf, gbuf, xbuf, ybuf, zbuf, tbuf, ibuf,
                  g0b0, g0b1, g1b0, g1b1, f0b0, f0b1, f1b0, f1b1, obx, oby, obz, omg,
                  semi0, semi1, semg0, semg1, semo0, semo1,
                  nsub, inv_h, om0):
    wid = lax.axis_index("s") * _NC + lax.axis_index("c")
    base = wid * (nsub * _CN)
    semi = (semi0, semi1)
    semg = (semg0, semg1)
    semo = (semo0, semo1)
    g0buf = (g0b0, g0b1)
    g1buf = (g1b0, g1b1)
    f0buf = (f0b0, f0b1)
    f1buf = (f1b0, f1b1)
    pltpu.sync_copy(omg_hbm, omg)

    def start_in(k, s):
        cs = pl.ds(base + k * _CN, _CN)
        return [
            pltpu.async_copy(om_hbm.at[cs], ombuf.at[s], semi[s]),
            pltpu.async_copy(grow_hbm.at[cs], gbuf.at[s], semi[s]),
            pltpu.async_copy(x_hbm.at[cs], xbuf.at[s], semi[s]),
            pltpu.async_copy(y_hbm.at[cs], ybuf.at[s], semi[s]),
            pltpu.async_copy(z_hbm.at[cs], zbuf.at[s], semi[s]),
        ]

    def start_out(k, s):
        cs = pl.ds(base + k * _CN, _CN)
        return [
            pltpu.async_copy(obx.at[s], ox_hbm.at[cs], semo[s]),
            pltpu.async_copy(oby.at[s], oy_hbm.at[cs], semo[s]),
            pltpu.async_copy(obz.at[s], oz_hbm.at[cs], semo[s]),
        ]

    def it1(s):
        def body(i, _):
            sl = pl.ds(i * 16, 16)
            om = ombuf[s, sl]
            grow = gbuf[s, sl]
            pos = (om - om0) * inv_h
            c0 = jnp.clip(pos.astype(jnp.int32), 0, _N_OM - 2)
            w0 = plsc.load_gather(omg, [c0])
            w1 = plsc.load_gather(omg, [c0 + 1])
            j = c0 + (w0 < om).astype(jnp.int32) + (w1 < om).astype(jnp.int32)
            c = jnp.clip(j, 1, _N_OM - 1)
            x1 = plsc.load_gather(omg, [c])
            x0 = plsc.load_gather(omg, [c - 1])
            g0 = grow + c - 1
            tbuf[s, sl] = (om - x0) / (x1 - x0)
            ibuf[s, sl] = 1.0 / om
            g0buf[s][sl] = g0
            g1buf[s][sl] = g0 + 1
            return 0
        lax.fori_loop(0, _CN // 16, body, 0)

    def it2(s):
        def body(i, _):
            sl = pl.ds(i * 16, 16)
            f0 = f0buf[s][sl]
            f1 = f1buf[s][sl]
            q = (f0 + (f1 - f0) * tbuf[s, sl]) * ibuf[s, sl]
            obx[s, sl] = q * xbuf[s, sl]
            oby[s, sl] = q * ybuf[s, sl]
            obz[s, sl] = q * zbuf[s, sl]
            return 0
        lax.fori_loop(0, _CN // 16, body, 0)

    in_h = [None, None]
    g_h = [None, None]
    o_h = [None, None]
    in_h[0] = start_in(0, 0)
    if nsub > 1:
        in_h[1] = start_in(1, 1)
    for k in range(nsub):
        s = k & 1
        s2 = 1 - s
        for h in in_h[s]:
            h.wait()
        it1(s)
        g_h[s] = [
            pltpu.async_copy(tab_hbm.at[g0buf[s]], f0buf[s], semg[s]),
            pltpu.async_copy(tab_hbm.at[g1buf[s]], f1buf[s], semg[s]),
        ]
        if k >= 1:
            for h in g_h[s2]:
                h.wait()
            if o_h[s2] is not None:
                for h in o_h[s2]:
                    h.wait()
            it2(s2)
            o_h[s2] = start_out(k - 1, s2)
            if k + 1 < nsub:
                in_h[s2] = start_in(k + 1, s2)
    s = (nsub - 1) & 1
    for h in g_h[s]:
        h.wait()
    if o_h[s] is not None:
        for h in o_h[s]:
            h.wait()
    it2(s)
    o_h[s] = start_out(nsub - 1, s)
    for ss in (0, 1):
        if o_h[ss] is not None:
            for h in o_h[ss]:
                h.wait()


def kernel(scale, vec, omegas_array, score_norms):
    b = scale.shape[0]
    assert b % (_NW * _CN) == 0
    nsub = b // (_NW * _CN)
    xs = vec[:, 0]
    ys = vec[:, 1]
    zs = vec[:, 2]
    tab_flat = score_norms.reshape(_N_EPS * _N_OM)

    grid = 16
    bs = b // grid
    om, grow = pl.pallas_call(
        _tc_body,
        grid=(grid,),
        in_specs=[pl.BlockSpec((bs,), lambda i: (i,))] * 4,
        out_specs=[pl.BlockSpec((bs,), lambda i: (i,))] * 2,
        out_shape=[
            jax.ShapeDtypeStruct((b,), jnp.float32),
            jax.ShapeDtypeStruct((b,), jnp.int32),
        ],
    )(scale, xs, ys, zs)

    h = (np.pi - 1e-3) / (_N_OM - 1)
    mesh = plsc.VectorSubcoreMesh(core_axis_name="c", subcore_axis_name="s")
    main = pl.kernel(
        functools.partial(_sc_main_body, nsub=nsub,
                          inv_h=np.float32(1.0 / h), om0=np.float32(1e-3)),
        out_type=[jax.ShapeDtypeStruct((b,), jnp.float32)] * 3,
        mesh=mesh,
        scratch_types=[
            pltpu.VMEM((2, _CN), jnp.float32),
            pltpu.VMEM((2, _CN), jnp.int32),
            pltpu.VMEM((2, _CN), jnp.float32),
            pltpu.VMEM((2, _CN), jnp.float32),
            pltpu.VMEM((2, _CN), jnp.float32),
            pltpu.VMEM((2, _CN), jnp.float32),
            pltpu.VMEM((2, _CN), jnp.float32),
            pltpu.VMEM((_CN,), jnp.int32),
            pltpu.VMEM((_CN,), jnp.int32),
            pltpu.VMEM((_CN,), jnp.int32),
            pltpu.VMEM((_CN,), jnp.int32),
            pltpu.VMEM((_CN,), jnp.float32),
            pltpu.VMEM((_CN,), jnp.float32),
            pltpu.VMEM((_CN,), jnp.float32),
            pltpu.VMEM((_CN,), jnp.float32),
            pltpu.VMEM((2, _CN), jnp.float32),
            pltpu.VMEM((2, _CN), jnp.float32),
            pltpu.VMEM((2, _CN), jnp.float32),
            pltpu.VMEM((_N_OM,), jnp.float32),
            pltpu.SemaphoreType.DMA,
            pltpu.SemaphoreType.DMA,
            pltpu.SemaphoreType.DMA,
            pltpu.SemaphoreType.DMA,
            pltpu.SemaphoreType.DMA,
            pltpu.SemaphoreType.DMA,
        ],
        compiler_params=pltpu.CompilerParams(needs_layout_passes=False),
    )
    ox, oy, oz = main(om, grow, xs, ys, zs, omegas_array, tab_flat)
    return jnp.stack([ox, oy, oz], axis=1).astype(scale.dtype)
